# baseline (speedup 1.0000x reference)
import functools

import jax
import jax.numpy as jnp
from jax import lax
from jax.experimental import pallas as pl
from jax.experimental.pallas import tpu as pltpu
from jax.experimental.pallas import tpu_sc as plsc

_V = 27012
_B = 16384
_L = 50
_LS = 24
_LD = _L - _LS
_LDP = 32
_LP = 64
_D = 768
_DP = _D // 2
_NG = _D // 32
_NC, _NS = 2, 16
_NW = _NC * _NS
_WPT = _B // _NW
_LANES = 16
_NBUF = 4
_NSEM = 4
_CH = 64
_NCHUNK = _WPT // _CH
_OG = 8
_GPC = _CH // _OG


def _sc_body(idxa_hbm, idxb_hbm, fre_hbm, tab2_hbm, tab1_hbm, out_hbm,
             idxa_v, idxb_v, fre_v, rows_s, rows_d, out_v, red_v,
             ssem, dsem, osem):
  wid = lax.axis_index("s") * _NC + lax.axis_index("c")
  base = wid * _WPT

  def stream_desc(w_local, buf):
    return pltpu.make_async_copy(
        tab2_hbm.at[idxa_v.at[w_local]],
        rows_s.at[buf],
        ssem.at[buf],
    )

  def gather_start(w_local, buf):
    stream_desc(w_local, buf).start()
    sbase = w_local * _LDP
    for g4 in range((_LD + _LANES - 1) // _LANES):
      iv = idxb_v[pl.ds(sbase + g4 * _LANES, _LANES)]
      for j in range(min(_LANES, _LD - g4 * _LANES)):
        srow = iv[j]
        l = g4 * _LANES + j
        pltpu.make_async_copy(
            tab1_hbm.at[pl.ds(srow * _DP, _DP)],
            rows_d.at[buf, pl.ds(l * _DP, _DP)],
            dsem.at[buf, l % _NSEM],
        ).start()

  def gather_wait(w_local, buf):
    stream_desc(w_local, buf).wait()
    for k in range(_NSEM):
      nrows = (_LD - k + _NSEM - 1) // _NSEM
      pltpu.make_async_copy(
          tab1_hbm.at[pl.ds(0, nrows * _DP)],
          rows_d.at[buf, pl.ds(0, nrows * _DP)],
          dsem.at[buf, k],
      ).wait()

  def out_desc(group_start, buf):
    return pltpu.make_async_copy(
        out_v.at[buf],
        out_hbm.at[pl.ds(group_start, _OG)],
        osem.at[buf],
    )

  def compute_word(wl, buf):
    woff = wl * _LP
    fsum = (fre_v[pl.ds(woff, _LANES)]
            + fre_v[pl.ds(woff + 16, _LANES)]
            + fre_v[pl.ds(woff + 32, _LANES)]
            + fre_v[pl.ds(woff + 48, _LANES)])
    lanes = lax.iota(jnp.int32, _LANES)
    for sh in (1, 2, 4, 8):
      red_v[...] = fsum
      fsum = fsum + plsc.load_gather(red_v, [lanes ^ sh])
    inv_vec = 1.0 / fsum
    ob = (wl // _OG) % 2
    slot = wl % _OG

    def wsplat(l):
      idxv = jnp.full((_LANES,), woff + l, jnp.int32)
      return plsc.load_gather(fre_v, [idxv]) * inv_vec

    def halves_s(l, g):
      v = rows_s[buf, l, pl.ds(g * _LANES, _LANES)]
      ab = plsc.bitcast(v, jnp.bfloat16)
      return plsc.unpack(ab, format=plsc.PackFormat.INTERLEAVED)

    def halves_d(l, g):
      v = rows_d[buf, pl.ds((l - _LS) * _DP + g * _LANES, _LANES)]
      ab = plsc.bitcast(v, jnp.bfloat16)
      return plsc.unpack(ab, format=plsc.PackFormat.INTERLEAVED)

    w0 = wsplat(0)
    for g in range(_NG):
      a, b2 = halves_s(0, g)
      out_v[ob, slot, pl.ds(g * 32, _LANES)] = a * w0
      out_v[ob, slot, pl.ds(g * 32 + 16, _LANES)] = b2 * w0

    @pl.loop(1, _LS)
    def _acc_s(l):
      w = wsplat(l)
      for g in range(_NG):
        a, b2 = halves_s(l, g)
        plsc.addupdate(out_v.at[ob, slot, pl.ds(g * 32, _LANES)], a * w)
        plsc.addupdate(out_v.at[ob, slot, pl.ds(g * 32 + 16, _LANES)], b2 * w)

    @pl.loop(_LS, _L)
    def _acc_d(l):
      w = wsplat(l)
      for g in range(_NG):
        a, b2 = halves_d(l, g)
        plsc.addupdate(out_v.at[ob, slot, pl.ds(g * 32, _LANES)], a * w)
        plsc.addupdate(out_v.at[ob, slot, pl.ds(g * 32 + 16, _LANES)], b2 * w)

  @pl.loop(0, _NCHUNK)
  def _chunk(ci):
    cstart = base + ci * _CH
    pltpu.sync_copy(idxa_hbm.at[pl.ds(cstart, _CH)], idxa_v)
    pltpu.sync_copy(idxb_hbm.at[pl.ds(cstart * _LDP, _CH * _LDP)],
                    idxb_v.at[pl.ds(0, _CH * _LDP)])
    pltpu.sync_copy(fre_hbm.at[pl.ds(cstart * _LP, _CH * _LP)], fre_v)
    for p in range(_NBUF - 1):
      gather_start(p, p)

    @pl.loop(0, _CH, step=_NBUF)
    def _words(wb):
      for b in range(_NBUF):
        wl = wb + b
        g_global = ci * _GPC + wl // _OG
        ob = (wl // _OG) % 2

        @pl.when(jnp.logical_and(wl % _OG == 0, g_global >= 2))
        def _():
          out_desc(base, ob).wait()

        gather_wait(wl, b)

        compute_word(wl, b)

        @pl.when(wl + _NBUF - 1 < _CH)
        def _():
          gather_start(wl + _NBUF - 1, (b + _NBUF - 1) % _NBUF)

        @pl.when(wl % _OG == _OG - 1)
        def _():
          out_desc(cstart + (wl // _OG) * _OG, ob).start()

  out_desc(base, 0).wait()
  out_desc(base, 1).wait()


@functools.partial(jax.jit, static_argnums=())
def _run(idxa, idxb, fre_flat, tab2, tab1):
  mesh = plsc.VectorSubcoreMesh(
      core_axis_name="c", subcore_axis_name="s",
      num_cores=_NC, num_subcores=_NS)
  k = pl.kernel(
      _sc_body,
      out_type=jax.ShapeDtypeStruct((_B, _D), jnp.float32),
      mesh=mesh,
      compiler_params=pltpu.CompilerParams(needs_layout_passes=False),
      scratch_types=[
          pltpu.VMEM((_CH, _LS), jnp.int32),
          pltpu.VMEM((_CH * _LDP + _LANES,), jnp.int32),
          pltpu.VMEM((_CH * _LP,), jnp.float32),
          pltpu.VMEM((_NBUF, _LS, _DP), jnp.float32),
          pltpu.VMEM((_NBUF, _LD * _DP), jnp.float32),
          pltpu.VMEM((2, _OG, _D), jnp.float32),
          pltpu.VMEM((_LANES,), jnp.float32),
          pltpu.SemaphoreType.DMA((_NBUF,)),
          pltpu.SemaphoreType.DMA((_NBUF, _NSEM)),
          pltpu.SemaphoreType.DMA((2,)),
      ],
  )
  return k(idxa, idxb, fre_flat, tab2, tab1)


def kernel(indices, fre, table):
  idx32 = indices.astype(jnp.int32)
  idxa = idx32[:, :_LS]
  idxb = jnp.pad(idx32[:, _LS:], ((0, 0), (0, _LDP - _LD))).reshape(-1)
  frep = jnp.pad(fre, ((0, 0), (0, _LP - _L)))
  tb = table.astype(jnp.bfloat16).reshape(_V, _NG, 2, _LANES)
  tcols = tb.transpose(0, 1, 3, 2)
  tpacked = lax.bitcast_convert_type(tcols, jnp.float32)
  return _run(idxa, idxb, frep.reshape(-1),
              tpacked.reshape(_V, _DP), tpacked.reshape(-1))


# speedup vs baseline: 1.0062x; 1.0062x over previous
import functools

import jax
import jax.numpy as jnp
from jax import lax
from jax.experimental import pallas as pl
from jax.experimental.pallas import tpu as pltpu
from jax.experimental.pallas import tpu_sc as plsc

_V = 27012
_B = 16384
_L = 50
_LP = 64
_LPI = 56
_D = 768
_DP = _D // 2
_NG = _D // 32
_NC, _NS = 2, 16
_NW = _NC * _NS
_WPT = _B // _NW
_LANES = 16
_NBUF = 4
_CH = 64
_NCHUNK = _WPT // _CH
_OG = 8
_GPC = _CH // _OG


_NSEM = 4


def _sc_body(idx_hbm, fre_hbm, table_hbm, out_hbm,
             idx_v, fre_v, rows_v, out_v, red_v, gsem, osem):
  wid = lax.axis_index("s") * _NC + lax.axis_index("c")
  base = wid * _WPT

  def gather_start(w_local, buf):
    sbase = w_local * _LPI
    for g4 in range((_L + _LANES - 1) // _LANES):
      iv = idx_v[pl.ds(sbase + g4 * _LANES, _LANES)]
      for j in range(min(_LANES, _L - g4 * _LANES)):
        srow = iv[j]
        l = g4 * _LANES + j
        pltpu.make_async_copy(
            table_hbm.at[pl.ds(srow * _DP, _DP)],
            rows_v.at[buf, pl.ds(l * _DP, _DP)],
            gsem.at[buf, l % _NSEM],
        ).start()

  def gather_wait(buf):
    for k in range(_NSEM):
      nrows = (_L - k + _NSEM - 1) // _NSEM
      pltpu.make_async_copy(
          table_hbm.at[pl.ds(0, nrows * _DP)],
          rows_v.at[buf, pl.ds(0, nrows * _DP)],
          gsem.at[buf, k],
      ).wait()

  def out_desc(group_start, buf):
    return pltpu.make_async_copy(
        out_v.at[buf],
        out_hbm.at[pl.ds(group_start, _OG)],
        osem.at[buf],
    )

  def compute_word(wl, buf):
    woff = wl * _LP
    fsum = (fre_v[pl.ds(woff, _LANES)]
            + fre_v[pl.ds(woff + 16, _LANES)]
            + fre_v[pl.ds(woff + 32, _LANES)]
            + fre_v[pl.ds(woff + 48, _LANES)])
    lanes = lax.iota(jnp.int32, _LANES)
    for sh in (1, 2, 4, 8):
      red_v[...] = fsum
      fsum = fsum + plsc.load_gather(red_v, [lanes ^ sh])
    inv_vec = 1.0 / fsum
    ob = (wl // _OG) % 2
    slot = wl % _OG

    def wsplat(l):
      idxv = jnp.full((_LANES,), woff + l, jnp.int32)
      return plsc.load_gather(fre_v, [idxv]) * inv_vec

    def row_halves(l, g):
      v = rows_v[buf, pl.ds(l * _DP + g * _LANES, _LANES)]
      ab = plsc.bitcast(v, jnp.bfloat16)
      return plsc.unpack(ab, format=plsc.PackFormat.INTERLEAVED)

    w0 = wsplat(0)
    for g in range(_NG):
      a, b2 = row_halves(0, g)
      out_v[ob, slot, pl.ds(g * 32, _LANES)] = a * w0
      out_v[ob, slot, pl.ds(g * 32 + 16, _LANES)] = b2 * w0

    @pl.loop(1, _L)
    def _acc(l):
      w = wsplat(l)
      for g in range(_NG):
        a, b2 = row_halves(l, g)
        plsc.addupdate(out_v.at[ob, slot, pl.ds(g * 32, _LANES)], a * w)
        plsc.addupdate(out_v.at[ob, slot, pl.ds(g * 32 + 16, _LANES)], b2 * w)

  @pl.loop(0, _NCHUNK)
  def _chunk(ci):
    cstart = base + ci * _CH
    pltpu.sync_copy(idx_hbm.at[pl.ds(cstart * _LPI, _CH * _LPI)],
                    idx_v.at[pl.ds(0, _CH * _LPI)])
    pltpu.sync_copy(fre_hbm.at[pl.ds(cstart * _LP, _CH * _LP)], fre_v)
    for p in range(_NBUF - 1):
      gather_start(p, p)

    @pl.loop(0, _CH, step=_NBUF)
    def _words(wb):
      for b in range(_NBUF):
        wl = wb + b
        g_global = ci * _GPC + wl // _OG
        ob = (wl // _OG) % 2

        @pl.when(jnp.logical_and(wl % _OG == 0, g_global >= 2))
        def _():
          out_desc(base, ob).wait()

        gather_wait(b)

        compute_word(wl, b)

        @pl.when(wl + _NBUF - 1 < _CH)
        def _():
          gather_start(wl + _NBUF - 1, (b + _NBUF - 1) % _NBUF)

        @pl.when(wl % _OG == _OG - 1)
        def _():
          out_desc(cstart + (wl // _OG) * _OG, ob).start()

  out_desc(base, 0).wait()
  out_desc(base, 1).wait()


@functools.partial(jax.jit, static_argnums=())
def _run(idx2d, fre_flat, table_packed):
  mesh = plsc.VectorSubcoreMesh(
      core_axis_name="c", subcore_axis_name="s",
      num_cores=_NC, num_subcores=_NS)
  k = pl.kernel(
      _sc_body,
      out_type=jax.ShapeDtypeStruct((_B, _D), jnp.float32),
      mesh=mesh,
      compiler_params=pltpu.CompilerParams(needs_layout_passes=False),
      scratch_types=[
          pltpu.VMEM((_CH * _LPI + _LANES,), jnp.int32),
          pltpu.VMEM((_CH * _LP,), jnp.float32),
          pltpu.VMEM((_NBUF, _L * _DP), jnp.float32),
          pltpu.VMEM((2, _OG, _D), jnp.float32),
          pltpu.VMEM((_LANES,), jnp.float32),
          pltpu.SemaphoreType.DMA((_NBUF, _NSEM)),
          pltpu.SemaphoreType.DMA((2,)),
      ],
  )
  return k(idx2d, fre_flat, table_packed)


def kernel(indices, fre, table):
  idxp = jnp.pad(indices.astype(jnp.int32), ((0, 0), (0, _LPI - _L))).reshape(-1)
  frep = jnp.pad(fre, ((0, 0), (0, _LP - _L)))
  tb = table.astype(jnp.bfloat16).reshape(_V, _NG, 2, _LANES)
  tcols = tb.transpose(0, 1, 3, 2)
  tpacked = lax.bitcast_convert_type(tcols, jnp.float32).reshape(-1)
  return _run(idxp, frep.reshape(-1), tpacked)
